# Initial kernel scaffold; baseline (speedup 1.0000x reference)
import jax
import jax.numpy as jnp
from jax.experimental import pallas as pl


def kernel(x, edge_index, params):
    raise NotImplementedError("write your pallas kernel here")


# speedup vs baseline: 1.0316x; 1.0316x over previous
import functools

import jax
import jax.numpy as jnp
from jax import lax
from jax.experimental import pallas as pl
from jax.experimental.pallas import tpu as pltpu
from jax.experimental.pallas import tpu_sc as plsc

N = 100000
E = 3200000
H = 128
OUT = 3
BN_EPS = 1e-5
BN = 2000
GRID = N // BN


def _finish(out, gs, b):
    nrm = jnp.sqrt(jnp.sum(out * out, axis=1, keepdims=True))
    out = out / jnp.maximum(nrm, 1e-12)
    out = out * gs + b
    return jnp.where(out > 0, out, jnp.exp(jnp.minimum(out, 0.0)) - 1.0)


def _dense0_body(part_ref, x_ref, wl_ref, bl_ref, wr_ref, gs_ref, b_ref,
                 h_ref, rcnt_ref):
    p = part_ref[0] + part_ref[1]
    cnt = p[:, 10:11]
    rcnt = 1.0 / jnp.maximum(cnt, 1.0)
    mean = p * rcnt
    out = (jnp.dot(mean, wl_ref[...], preferred_element_type=jnp.float32)
           + jnp.dot(x_ref[...], wr_ref[...], preferred_element_type=jnp.float32)
           + bl_ref[...])
    h_ref[...] = _finish(out, gs_ref[...], b_ref[...])
    rcnt_ref[...] = rcnt


def _dense0(partials, x16, wl, bl, wr, gs, b):
    row = lambda i: (i, 0)
    full = lambda i: (0, 0)
    return pl.pallas_call(
        _dense0_body,
        grid=(GRID,),
        in_specs=[
            pl.BlockSpec((2, BN, 16), lambda i: (0, i, 0)),
            pl.BlockSpec((BN, 16), row),
            pl.BlockSpec((16, H), full),
            pl.BlockSpec((1, H), full),
            pl.BlockSpec((16, H), full),
            pl.BlockSpec((1, H), full),
            pl.BlockSpec((1, H), full),
        ],
        out_specs=[pl.BlockSpec((BN, H), row), pl.BlockSpec((BN, 1), row)],
        out_shape=[jax.ShapeDtypeStruct((N, H), jnp.float32),
                   jax.ShapeDtypeStruct((N, 1), jnp.float32)],
    )(partials, x16, wl, bl, wr, gs, b)


def _dense_body(with_head, agg_ref, h_ref, rcnt_ref, wl_ref, bl_ref, wr_ref,
                gs_ref, b_ref, *rest):
    out_ref = rest[-1]
    mean = agg_ref[...] * rcnt_ref[...]
    out = (jnp.dot(mean, wl_ref[...], preferred_element_type=jnp.float32)
           + jnp.dot(h_ref[...], wr_ref[...], preferred_element_type=jnp.float32)
           + bl_ref[...])
    out = _finish(out, gs_ref[...], b_ref[...])
    if with_head:
        (w0, b0, gs0, be0, w1, b1, gs1, be1, w2, b2) = rest[:-1]
        out = jnp.dot(out, w0[...], preferred_element_type=jnp.float32) + b0[...]
        out = jnp.maximum(out * gs0[...] + be0[...], 0.0)
        out = jnp.dot(out, w1[...], preferred_element_type=jnp.float32) + b1[...]
        out = jnp.maximum(out * gs1[...] + be1[...], 0.0)
        out = jnp.dot(out, w2[...], preferred_element_type=jnp.float32) + b2[...]
    out_ref[...] = out


def _dense(agg, h, rcnt, wl, bl, wr, gs, b, head=None):
    row = lambda i: (i, 0)
    full = lambda i: (0, 0)
    args = [agg, h, rcnt, wl, bl, wr, gs, b]
    in_specs = [
        pl.BlockSpec((BN, H), row),
        pl.BlockSpec((BN, H), row),
        pl.BlockSpec((BN, 1), row),
        pl.BlockSpec((H, H), full),
        pl.BlockSpec((1, H), full),
        pl.BlockSpec((H, H), full),
        pl.BlockSpec((1, H), full),
        pl.BlockSpec((1, H), full),
    ]
    if head is not None:
        args += list(head)
        in_specs += [pl.BlockSpec((H, H), full) if a.shape == (H, H)
                     else pl.BlockSpec((1, H), full) for a in head]
    return pl.pallas_call(
        functools.partial(_dense_body, head is not None),
        grid=(GRID,),
        in_specs=in_specs,
        out_specs=pl.BlockSpec((BN, H), row),
        out_shape=jax.ShapeDtypeStruct((N, H), jnp.float32),
    )(*args)


def _agg16_partials(x16, src, dst):
    agg = jax.ops.segment_sum(jnp.take(x16, src, axis=0), dst, num_segments=N)
    return jnp.stack([agg, jnp.zeros_like(agg)])


def _agg128(h, src, dst):
    return jax.ops.segment_sum(jnp.take(h, src, axis=0), dst, num_segments=N)


def kernel(x, edge_index, params):
    src = edge_index[0]
    dst = edge_index[1]
    x16 = jnp.zeros((N, 16), jnp.float32).at[:, :10].set(x).at[:, 10].set(1.0)

    def r(v):
        return jnp.reshape(v, (1, H))

    sc = 1.0 / jnp.sqrt(1.0 + BN_EPS)
    wl0 = jnp.zeros((16, H), jnp.float32).at[:10].set(params["conv0_Wl"])
    wr0 = jnp.zeros((16, H), jnp.float32).at[:10].set(params["conv0_Wr"])
    partials = _agg16_partials(x16, src, dst)
    h, rcnt = _dense0(partials, x16, wl0, r(params["conv0_bl"]), wr0,
                      r(params["bn0_g"] * sc), r(params["bn0_b"]))

    head = (params["p0_W"], r(params["p0_b"]), r(params["p0_g"] * sc),
            r(params["p0_be"]),
            params["p1_W"], r(params["p1_b"]), r(params["p1_g"] * sc),
            r(params["p1_be"]),
            jnp.zeros((H, H), jnp.float32).at[:, :OUT].set(params["p2_W"]),
            jnp.zeros((1, H), jnp.float32).at[0, :OUT].set(params["p2_b"]))

    for i in range(1, 5):
        agg = _agg128(h, src, dst)
        h = _dense(agg, h, rcnt,
                   params[f"conv{i}_Wl"], r(params[f"conv{i}_bl"]),
                   params[f"conv{i}_Wr"],
                   r(params[f"bn{i}_g"] * sc), r(params[f"bn{i}_b"]),
                   head=head if i == 4 else None)
    return h[:, :OUT]


# speedup vs baseline: 2.4945x; 2.4182x over previous
import functools

import jax
import jax.numpy as jnp
from jax import lax
from jax.experimental import pallas as pl
from jax.experimental.pallas import tpu as pltpu
from jax.experimental.pallas import tpu_sc as plsc

N = 100000
NP = 100096
E = 3200000
H = 128
OUT = 3
BN_EPS = 1e-5
BN = 3128
GRID = NP // BN

NC, NS = 2, 16
CHN = 12512
ACC = 12544
DUMP = 12512
NPA = NP
BE = 2000
NBLK = (E // NS) // BE
FCAP = BE + 48
_MESH = plsc.VectorSubcoreMesh(core_axis_name="c", subcore_axis_name="s")


def _sc_agg_body(h_hbm, src_hbm, dst_hbm, agg_hbm,
                 acc, sbuf, dbuf, csrcf, cdstf, g32s, g32d, rows):
    c = lax.axis_index("c")
    s = lax.axis_index("s")

    ept = E // NS
    ebase = s * ept

    for j in range(4):
        chunk = 2 * j + c
        lo = chunk * CHN

        @pl.loop(0, 32)
        def _zr(i):
            for q in range(8):
                rows[i, pl.ds(q * 16, 16)] = jnp.zeros((16,), jnp.float32)

        @pl.when(s < 8)
        def _():
            @pl.loop(0, 49)
            def _zc(k):
                pltpu.sync_copy(rows, acc.at[pl.ds(s * 1568 + k * 32, 32)])
        plsc.subcore_barrier()

        @pl.loop(0, NBLK)
        def _blk(i):
            off = ebase + i * BE
            pltpu.sync_copy(src_hbm.at[pl.ds(off, BE)], sbuf)
            pltpu.sync_copy(dst_hbm.at[pl.ds(off, BE)], dbuf)

            @pl.loop(0, BE // 16, init_carry=jnp.int32(0))
            def _vec(v, cur):
                sv = sbuf[pl.ds(v * 16, 16)]
                dv = dbuf[pl.ds(v * 16, 16)]
                m = (dv >= lo) & (dv < lo + CHN)
                plsc.store_compressed(csrcf.at[pl.ds(cur, 16)], sv, mask=m)
                plsc.store_compressed(cdstf.at[pl.ds(cur, 16)], dv - lo, mask=m)
                return cur + jnp.sum(jnp.where(m, 1, 0).astype(jnp.int32))

            cur = _vec
            zv = jnp.zeros((16,), jnp.int32)
            csrcf[pl.ds(cur, 16)] = zv
            csrcf[pl.ds(cur + 16, 16)] = zv
            cdstf[pl.ds(cur, 16)] = zv + DUMP
            cdstf[pl.ds(cur + 16, 16)] = zv + DUMP

            @pl.loop(0, (cur + 31) // 32)
            def _flush(q):
                for t in range(2):
                    g32s[pl.ds(t * 16, 16)] = csrcf[pl.ds(q * 32 + t * 16, 16)]
                    g32d[pl.ds(t * 16, 16)] = cdstf[pl.ds(q * 32 + t * 16, 16)]
                pltpu.sync_copy(h_hbm.at[g32s], rows)
                pltpu.sync_copy(rows, acc.at[g32d], add=True)

        plsc.subcore_barrier()

        @pl.when(s < 7)
        def _():
            pltpu.sync_copy(acc.at[pl.ds(s * 1568, 1568)],
                            agg_hbm.at[pl.ds(lo + s * 1568, 1568)])
        @pl.when(s == 7)
        def _():
            pltpu.sync_copy(acc.at[pl.ds(7 * 1568, 1536)],
                            agg_hbm.at[pl.ds(lo + 7 * 1568, 1536)])
        plsc.subcore_barrier()


def _agg128(h, src, dst):
    return pl.kernel(
        _sc_agg_body,
        out_type=jax.ShapeDtypeStruct((NPA, H), jnp.float32),
        mesh=_MESH,
        scratch_types=[
            pltpu.VMEM_SHARED((ACC, H), jnp.float32),
            pltpu.VMEM((BE,), jnp.int32),
            pltpu.VMEM((BE,), jnp.int32),
            pltpu.VMEM((FCAP,), jnp.int32),
            pltpu.VMEM((FCAP,), jnp.int32),
            pltpu.VMEM((32,), jnp.int32),
            pltpu.VMEM((32,), jnp.int32),
            pltpu.VMEM((32, H), jnp.float32),
        ],
        compiler_params=pltpu.CompilerParams(needs_layout_passes=False),
    )(h, src, dst)


def _finish(out, gs, b):
    nrm = jnp.sqrt(jnp.sum(out * out, axis=1, keepdims=True))
    out = out / jnp.maximum(nrm, 1e-12)
    out = out * gs + b
    return jnp.where(out > 0, out, jnp.exp(jnp.minimum(out, 0.0)) - 1.0)


def _dense_body(mode, agg_ref, h_ref, rcnt_ref, wl_ref, bl_ref, wr_ref,
                gs_ref, b_ref, *rest):
    if mode == 0:
        out_ref, rcnt_out = rest[-2], rest[-1]
        cnt = agg_ref[:, 10:11]
        rcnt = 1.0 / jnp.maximum(cnt, 1.0)
        rcnt_out[...] = rcnt
    else:
        out_ref = rest[-1]
        rcnt = rcnt_ref[...]
    mean = agg_ref[...] * rcnt
    out = (jnp.dot(mean, wl_ref[...], preferred_element_type=jnp.float32)
           + jnp.dot(h_ref[...], wr_ref[...], preferred_element_type=jnp.float32)
           + bl_ref[...])
    out = _finish(out, gs_ref[...], b_ref[...])
    if mode == 2:
        (w0, b0, gs0, be0, w1, b1, gs1, be1, w2, b2) = rest[:-1]
        out = jnp.dot(out, w0[...], preferred_element_type=jnp.float32) + b0[...]
        out = jnp.maximum(out * gs0[...] + be0[...], 0.0)
        out = jnp.dot(out, w1[...], preferred_element_type=jnp.float32) + b1[...]
        out = jnp.maximum(out * gs1[...] + be1[...], 0.0)
        out = jnp.dot(out, w2[...], preferred_element_type=jnp.float32) + b2[...]
    out_ref[...] = out


def _dense(mode, agg, h, rcnt, wl, bl, wr, gs, b, head=()):
    row = lambda i: (i, 0)
    full = lambda i: (0, 0)
    args = [agg, h, rcnt, wl, bl, wr, gs, b] + list(head)
    in_specs = [
        pl.BlockSpec((BN, H), row),
        pl.BlockSpec((BN, H), row),
        pl.BlockSpec((BN, 1), row),
        pl.BlockSpec((H, H), full),
        pl.BlockSpec((1, H), full),
        pl.BlockSpec((H, H), full),
        pl.BlockSpec((1, H), full),
        pl.BlockSpec((1, H), full),
    ] + [pl.BlockSpec((H, H), full) if a.shape == (H, H)
         else pl.BlockSpec((1, H), full) for a in head]
    out_specs = [pl.BlockSpec((BN, H), row)]
    out_shape = [jax.ShapeDtypeStruct((NP, H), jnp.float32)]
    if mode == 0:
        out_specs.append(pl.BlockSpec((BN, 1), row))
        out_shape.append(jax.ShapeDtypeStruct((NP, 1), jnp.float32))
    res = pl.pallas_call(
        functools.partial(_dense_body, mode),
        grid=(GRID,),
        in_specs=in_specs,
        out_specs=out_specs,
        out_shape=out_shape,
    )(*args)
    return res if mode == 0 else res[0]


def kernel(x, edge_index, params):
    src = edge_index[0]
    dst = edge_index[1]
    h0 = jnp.zeros((NP, H), jnp.float32).at[:N, :10].set(x).at[:, 10].set(1.0)

    def r(v):
        return jnp.reshape(v, (1, H))

    def wpad(w):
        return jnp.zeros((H, H), jnp.float32).at[:w.shape[0], :w.shape[1]].set(w)

    sc = 1.0 / jnp.sqrt(1.0 + BN_EPS)
    rcnt0 = jnp.zeros((NP, 1), jnp.float32)

    agg = _agg128(h0, src, dst)
    h, rcnt = _dense(0, agg, h0, rcnt0,
                     wpad(params["conv0_Wl"]), r(params["conv0_bl"]),
                     wpad(params["conv0_Wr"]),
                     r(params["bn0_g"] * sc), r(params["bn0_b"]))

    head = (params["p0_W"], r(params["p0_b"]), r(params["p0_g"] * sc),
            r(params["p0_be"]),
            params["p1_W"], r(params["p1_b"]), r(params["p1_g"] * sc),
            r(params["p1_be"]),
            wpad(params["p2_W"]),
            jnp.zeros((1, H), jnp.float32).at[0, :OUT].set(params["p2_b"]))

    for i in range(1, 5):
        agg = _agg128(h, src, dst)
        h = _dense(2 if i == 4 else 1, agg, h, rcnt,
                   params[f"conv{i}_Wl"], r(params[f"conv{i}_bl"]),
                   params[f"conv{i}_Wr"],
                   r(params[f"bn{i}_g"] * sc), r(params[f"bn{i}_b"]),
                   head=head if i == 4 else ())
    return h[:N, :OUT]
